# baseline (speedup 1.0000x reference)
import jax
import jax.numpy as jnp
from jax.experimental import pallas as pl
from jax.experimental.pallas import tpu as pltpu

_PAD = 1
_SBLK = 512
_SUB = 512


def _body(tokv_ref, toks_ref, pad_ref, w_hbm, out_hbm, buf0, buf1, rsem0,
          rsem1, ws0, ws1, ws2, ws3, fsem):
    seq_len, bsz = tokv_ref.shape
    dim = pad_ref.shape[0]
    nstep = seq_len // _SBLK
    bufs = (buf0, buf1)
    rsems = (rsem0, rsem1)
    wsems = (ws0, ws1, ws2, ws3)

    def read_cp(j):
        return pltpu.make_async_copy(
            w_hbm.at[pl.ds((2 + j * _SBLK) * dim, _SBLK * dim)],
            bufs[j % 2], rsems[j % 2])

    def write_cp(j, b):
        return pltpu.make_async_copy(
            bufs[j % 2],
            out_hbm.at[pl.ds((b * seq_len + j * _SBLK) * dim, _SBLK * dim)],
            wsems[b])

    read_cp(0).start()
    for j in range(nstep):
        read_cp(j).wait()
        for b in range(bsz):
            write_cp(j, b).start()
        if j + 1 < nstep:
            if j >= 1:
                for b in range(bsz):
                    write_cp(j - 1, b).wait()
            read_cp(j + 1).start()
    for j in (nstep - 2, nstep - 1):
        for b in range(bsz):
            write_cp(j, b).wait()

    for b in range(bsz):
        for k in range(seq_len // _SUB):
            cnt = jnp.sum(
                (tokv_ref[pl.ds(k * _SUB, _SUB), pl.ds(b, 1)] == _PAD)
                .astype(jnp.int32))

            @pl.when(cnt > 0)
            def _fix(b=b, k=k):
                def fix_row(s, carry):
                    tok = toks_ref[b * seq_len + k * _SUB + s]

                    @pl.when(tok == _PAD)
                    def _():
                        cp = pltpu.make_async_copy(
                            pad_ref,
                            out_hbm.at[pl.ds(
                                (b * seq_len + k * _SUB + s) * dim, dim)],
                            fsem)
                        cp.start()
                        cp.wait()
                    return carry

                jax.lax.fori_loop(0, _SUB, fix_row, 0)


def kernel(input, weights):
    bsz, seq_len = input.shape
    dim = weights.shape[1]
    tokT = input.T
    pad_flat = jax.lax.slice(weights, (_PAD, 0), (_PAD + 1, dim)).reshape(-1)
    w_flat = weights.reshape(-1)
    out = pl.pallas_call(
        _body,
        in_specs=[
            pl.BlockSpec(memory_space=pltpu.VMEM),
            pl.BlockSpec(memory_space=pltpu.SMEM),
            pl.BlockSpec(memory_space=pltpu.VMEM),
            pl.BlockSpec(memory_space=pl.ANY),
        ],
        out_specs=pl.BlockSpec(memory_space=pl.ANY),
        out_shape=jax.ShapeDtypeStruct((bsz * seq_len * dim,), jnp.float32),
        scratch_shapes=[
            pltpu.VMEM((_SBLK * dim,), jnp.float32),
            pltpu.VMEM((_SBLK * dim,), jnp.float32),
            pltpu.SemaphoreType.DMA,
            pltpu.SemaphoreType.DMA,
            pltpu.SemaphoreType.DMA,
            pltpu.SemaphoreType.DMA,
            pltpu.SemaphoreType.DMA,
            pltpu.SemaphoreType.DMA,
            pltpu.SemaphoreType.DMA,
        ],
    )(tokT, input.reshape(-1), pad_flat, w_flat)
    return out.reshape(bsz, seq_len, dim)


# speedup vs baseline: 4.6515x; 4.6515x over previous
import jax
import jax.numpy as jnp
from jax.experimental import pallas as pl
from jax.experimental.pallas import tpu as pltpu

_PAD = 1
_SBLK = 1024


def _body(tokT_ref, pad_ref, wa_ref, wb_ref, out_ref):
    j = pl.program_id(0)
    w = jnp.concatenate(
        [wa_ref[pl.ds(2, _SBLK - 2), :], wb_ref[pl.ds(0, 2), :]], axis=0)
    pad = pad_ref[...]
    bsz = tokT_ref.shape[1]
    for b in range(bsz):
        mask = tokT_ref[pl.ds(j * _SBLK, _SBLK), pl.ds(b, 1)] != _PAD
        out_ref[b, :, :] = jnp.where(mask, w, pad)


def kernel(input, weights):
    bsz, seq_len = input.shape
    dim = weights.shape[1]
    pad_row = jax.lax.slice(weights, (_PAD, 0), (_PAD + 1, dim))
    tokT = input.T
    grid = (seq_len // _SBLK,)
    out = pl.pallas_call(
        _body,
        grid=grid,
        in_specs=[
            pl.BlockSpec((seq_len, bsz), lambda j: (0, 0)),
            pl.BlockSpec((1, dim), lambda j: (0, 0)),
            pl.BlockSpec((_SBLK, dim), lambda j: (j, 0)),
            pl.BlockSpec((8, dim), lambda j: ((j + 1) * (_SBLK // 8), 0)),
        ],
        out_specs=pl.BlockSpec((bsz, _SBLK, dim), lambda j: (0, j, 0)),
        out_shape=jax.ShapeDtypeStruct((bsz, seq_len, dim), jnp.float32),
    )(tokT, pad_row, weights, weights)
    return out
